# Initial kernel scaffold; baseline (speedup 1.0000x reference)
import jax
import jax.numpy as jnp
from jax.experimental import pallas as pl


def kernel(x, s_memory, m_memory, l_memory, s_ptr):
    raise NotImplementedError("write your pallas kernel here")


# speedup vs baseline: 2.4680x; 2.4680x over previous
import jax
import jax.numpy as jnp
from jax.experimental import pallas as pl
from jax.experimental.pallas import tpu as pltpu

DIM = 128
S_SIZE = 1024
M_SIZE = 8192
L_SIZE = 65536
BLK = 2048
M_BLOCKS = M_SIZE // BLK
L_BLOCKS = L_SIZE // BLK
N_FLASH = 1 + M_BLOCKS + L_BLOCKS
N_UTIL = M_BLOCKS + L_BLOCKS


def _flash_kernel(sptr_ref, x_ref, s_ref, m_ref, l_ref,
                  s_new_ref, out_ref, lse_ref,
                  acc_ref, mx_ref, den_ref, dbl_ref):
    i = pl.program_id(0)
    b = x_ref.shape[0]
    scale = 1.0 / jnp.sqrt(jnp.float32(DIM))
    x = x_ref[...]

    def flash_update(blk):
        scores = jax.lax.dot_general(
            x, blk, (((1,), (1,)), ((), ())),
            preferred_element_type=jnp.float32) * scale
        m_prev = mx_ref[...]
        m_new = jnp.maximum(m_prev, jnp.max(scores, axis=1, keepdims=True))
        alpha = jnp.exp(m_prev - m_new)
        p = jnp.exp(scores - m_new)
        den_ref[...] = den_ref[...] * alpha + jnp.sum(p, axis=1, keepdims=True)
        acc_ref[...] = acc_ref[...] * alpha + jax.lax.dot_general(
            p, blk, (((1,), (0,)), ((), ())),
            preferred_element_type=jnp.float32)
        mx_ref[...] = m_new

    @pl.when(i == 0)
    def _():
        sp = jax.lax.rem(sptr_ref[0], S_SIZE)
        sp = jnp.where(sp < 0, sp + S_SIZE, sp)
        xpad = jnp.concatenate(
            [x, jnp.zeros((S_SIZE - b, DIM), jnp.float32)], axis=0)
        dbl_ref[...] = jnp.concatenate([xpad, xpad], axis=0)
        rolled = dbl_ref[pl.ds(S_SIZE - sp, S_SIZE), :]
        r = jax.lax.broadcasted_iota(jnp.int32, (S_SIZE, 1), 0)
        off = jax.lax.rem(r - sp + 2 * S_SIZE, S_SIZE)
        written = off < b
        s_new = jnp.where(written, rolled, s_ref[...])
        s_new_ref[...] = s_new
        mx_ref[...] = jnp.full((b, 1), -jnp.inf, jnp.float32)
        den_ref[...] = jnp.zeros((b, 1), jnp.float32)
        acc_ref[...] = jnp.zeros((b, DIM), jnp.float32)
        flash_update(s_new)

    @pl.when(jnp.logical_and(i >= 1, i <= M_BLOCKS))
    def _():
        flash_update(m_ref[...])

    @pl.when(i > M_BLOCKS)
    def _():
        flash_update(l_ref[...])

    @pl.when(i == N_FLASH - 1)
    def _():
        den = den_ref[...]
        out_ref[...] = acc_ref[...] / den
        lse_ref[...] = mx_ref[...] + jnp.log(den)


def _util_kernel(x_ref, lse_ref, m_ref, l_ref, mu_ref, lu_ref):
    i = pl.program_id(0)
    scale = 1.0 / jnp.sqrt(jnp.float32(DIM))
    blk = jnp.where(i < M_BLOCKS, m_ref[...], l_ref[...])
    scores = jax.lax.dot_general(
        x_ref[...], blk, (((1,), (1,)), ((), ())),
        preferred_element_type=jnp.float32) * scale
    u = jnp.sum(jnp.exp(scores - lse_ref[...]), axis=0, keepdims=True)

    @pl.when(i < M_BLOCKS)
    def _():
        mu_ref[...] = u[None]

    @pl.when(i >= M_BLOCKS)
    def _():
        lu_ref[...] = u[None]


def kernel(x, s_memory, m_memory, l_memory, s_ptr):
    b = x.shape[0]
    sptr_arr = jnp.asarray(s_ptr, jnp.int32).reshape((1,))

    full = lambda shape: pl.BlockSpec(shape, lambda i: (0, 0))
    m_spec = pl.BlockSpec(
        (BLK, DIM), lambda i: (jnp.clip(i - 1, 0, M_BLOCKS - 1), 0))
    l_spec = pl.BlockSpec(
        (BLK, DIM), lambda i: (jnp.clip(i - 1 - M_BLOCKS, 0, L_BLOCKS - 1), 0))

    s_new, out, lse = pl.pallas_call(
        _flash_kernel,
        grid=(N_FLASH,),
        in_specs=[
            pl.BlockSpec(memory_space=pltpu.SMEM),
            full((b, DIM)),
            full((S_SIZE, DIM)),
            m_spec,
            l_spec,
        ],
        out_specs=[
            full((S_SIZE, DIM)),
            full((b, DIM)),
            full((b, 1)),
        ],
        out_shape=[
            jax.ShapeDtypeStruct((S_SIZE, DIM), jnp.float32),
            jax.ShapeDtypeStruct((b, DIM), jnp.float32),
            jax.ShapeDtypeStruct((b, 1), jnp.float32),
        ],
        scratch_shapes=[
            pltpu.VMEM((b, DIM), jnp.float32),
            pltpu.VMEM((b, 1), jnp.float32),
            pltpu.VMEM((b, 1), jnp.float32),
            pltpu.VMEM((2 * S_SIZE, DIM), jnp.float32),
        ],
    )(sptr_arr, x, s_memory, m_memory, l_memory)

    um_spec = pl.BlockSpec(
        (BLK, DIM), lambda i: (jnp.clip(i, 0, M_BLOCKS - 1), 0))
    ul_spec = pl.BlockSpec(
        (BLK, DIM), lambda i: (jnp.clip(i - M_BLOCKS, 0, L_BLOCKS - 1), 0))
    mu, lu = pl.pallas_call(
        _util_kernel,
        grid=(N_UTIL,),
        in_specs=[full((b, DIM)), full((b, 1)), um_spec, ul_spec],
        out_specs=[
            pl.BlockSpec((1, 1, BLK),
                         lambda i: (jnp.clip(i, 0, M_BLOCKS - 1), 0, 0)),
            pl.BlockSpec((1, 1, BLK),
                         lambda i: (jnp.clip(i - M_BLOCKS, 0, L_BLOCKS - 1), 0, 0)),
        ],
        out_shape=[
            jax.ShapeDtypeStruct((M_BLOCKS, 1, BLK), jnp.float32),
            jax.ShapeDtypeStruct((L_BLOCKS, 1, BLK), jnp.float32),
        ],
    )(x, lse, m_memory, l_memory)

    return out, s_new, mu.reshape(M_SIZE), lu.reshape(L_SIZE)
